# Initial kernel scaffold; baseline (speedup 1.0000x reference)
import jax
import jax.numpy as jnp
from jax.experimental import pallas as pl


def kernel(x, action_emb, action_pos_emb):
    raise NotImplementedError("write your pallas kernel here")


# speedup vs baseline: 3.4234x; 3.4234x over previous
import functools

import jax
import jax.numpy as jnp
from jax import lax
from jax.experimental import pallas as pl
from jax.experimental.pallas import tpu as pltpu
from jax.experimental.pallas import tpu_sc as plsc

D = 64
A = 24
NC, NS = 2, 16
NW = NC * NS
CHUNK = 768
GROW = 128
NG = CHUNK // GROW


def _make_sc_gather(B):
    b_per_w = B // NW
    n_chunks = b_per_w // CHUNK
    mesh = plsc.VectorSubcoreMesh(core_axis_name="c", subcore_axis_name="s")

    @functools.partial(
        pl.kernel,
        out_type=jax.ShapeDtypeStruct((B, D), jnp.float32),
        mesh=mesh,
        compiler_params=pltpu.CompilerParams(use_tc_tiling_on_sc=False),
        scratch_types=[
            pltpu.VMEM((b_per_w // GROW, GROW), jnp.int32),
            pltpu.VMEM((CHUNK, D), jnp.float32),
            pltpu.VMEM((A, D), jnp.float32),
            pltpu.SemaphoreType.DMA,
        ],
    )
    def body(x2d_hbm, emb_hbm, pos_hbm, out_hbm, idx_v, rows_v, pos_v, sem):
        wid = lax.axis_index("s") * NC + lax.axis_index("c")
        base = pl.multiple_of(wid * b_per_w, b_per_w)
        pltpu.sync_copy(pos_hbm.at[pl.ds(0, A)], pos_v)
        pltpu.sync_copy(
            x2d_hbm.at[pl.ds(pl.multiple_of(base // GROW, 8), b_per_w // GROW)],
            idx_v)

        def chunk_body(c, carry):
            row0 = pl.multiple_of(base + c * CHUNK, CHUNK)
            cps = [
                pltpu.async_copy(
                    emb_hbm.at[idx_v.at[c * NG + j]],
                    rows_v.at[pl.ds(j * GROW, GROW)],
                    sem,
                )
                for j in range(NG)
            ]
            for cp in cps:
                cp.wait()
            for u in range(A):
                pvecs = [pos_v[u, pl.ds(16 * l, 16)] for l in range(4)]

                def add_body(g, _, u=u, pvecs=pvecs):
                    r = g * A + u
                    for l in range(4):
                        plsc.addupdate(rows_v.at[r, pl.ds(16 * l, 16)], pvecs[l])
                    return 0

                lax.fori_loop(0, CHUNK // A, add_body, 0)
            pltpu.sync_copy(rows_v, out_hbm.at[pl.ds(row0, CHUNK)])
            return carry

        lax.fori_loop(0, n_chunks, chunk_body, 0)

    return body


def kernel(x, action_emb, action_pos_emb):
    B = x.size
    x2d = x.reshape(B // GROW, GROW)
    out = _make_sc_gather(B)(x2d, action_emb, action_pos_emb)
    return out.reshape(*x.shape, D)


# speedup vs baseline: 3.5075x; 1.0246x over previous
import functools

import jax
import jax.numpy as jnp
from jax import lax
from jax.experimental import pallas as pl
from jax.experimental.pallas import tpu as pltpu
from jax.experimental.pallas import tpu_sc as plsc

D = 64
A = 24
NC, NS = 2, 16
NW = NC * NS
CHUNK = 384
GROW = 128
NG = CHUNK // GROW
NBUF = 4
LAG = 2


def _make_sc_gather(B):
    b_per_w = B // NW
    n_chunks = b_per_w // CHUNK
    n_steps = n_chunks // NBUF
    mesh = plsc.VectorSubcoreMesh(core_axis_name="c", subcore_axis_name="s")

    @functools.partial(
        pl.kernel,
        out_type=jax.ShapeDtypeStruct((B, D), jnp.float32),
        mesh=mesh,
        compiler_params=pltpu.CompilerParams(use_tc_tiling_on_sc=False),
        scratch_types=[
            pltpu.VMEM((b_per_w // GROW, GROW), jnp.int32),
            [pltpu.VMEM((CHUNK, D), jnp.float32) for _ in range(NBUF)],
            pltpu.VMEM((A, D), jnp.float32),
            [pltpu.SemaphoreType.DMA for _ in range(NBUF)],
            [pltpu.SemaphoreType.DMA for _ in range(NBUF)],
        ],
    )
    def body(x2d_hbm, emb_hbm, pos_hbm, out_hbm, idx_v, rows, pos_v, gsem, osem):
        wid = lax.axis_index("s") * NC + lax.axis_index("c")
        base = pl.multiple_of(wid * b_per_w, b_per_w)
        pltpu.sync_copy(pos_hbm.at[pl.ds(0, A)], pos_v)
        pltpu.sync_copy(
            x2d_hbm.at[pl.ds(pl.multiple_of(base // GROW, 8), b_per_w // GROW)],
            idx_v)

        def fire_gather(c, b):
            for j in range(NG):
                pltpu.async_copy(
                    emb_hbm.at[idx_v.at[c * NG + j]],
                    rows[b].at[pl.ds(j * GROW, GROW)],
                    gsem[b])

        def wait_gather(b):
            pltpu.make_async_copy(out_hbm.at[pl.ds(0, CHUNK)], rows[b],
                                  gsem[b]).wait()

        def fire_out(c, b):
            row0 = pl.multiple_of(base + c * CHUNK, CHUNK)
            pltpu.async_copy(rows[b], out_hbm.at[pl.ds(row0, CHUNK)], osem[b])

        def wait_out(b):
            pltpu.make_async_copy(out_hbm.at[pl.ds(0, CHUNK)], rows[b],
                                  osem[b]).wait()

        def add_pos(b):
            for u in range(A):
                pvecs = [pos_v[u, pl.ds(16 * l, 16)] for l in range(4)]

                def add_body(g, _, b=b, u=u, pvecs=pvecs):
                    r = g * A + u
                    for l in range(4):
                        plsc.addupdate(rows[b].at[r, pl.ds(16 * l, 16)],
                                       pvecs[l])
                    return 0

                lax.fori_loop(0, CHUNK // A, add_body, 0)

        for b in range(NBUF):
            fire_gather(b, b)

        def step_body(s, carry):
            for k in range(NBUF):
                c = s * NBUF + k
                wait_gather(k)
                add_pos(k)
                fire_out(c, k)
                m = c - LAG
                bm = (k - LAG) % NBUF

                @pl.when(jnp.logical_and(m >= 0, m + NBUF < n_chunks))
                def _(m=m, bm=bm):
                    wait_out(bm)
                    fire_gather(m + NBUF, bm)

            return carry

        lax.fori_loop(0, n_steps, step_body, 0)
        for b in range(NBUF):
            wait_out(b)

    return body


def kernel(x, action_emb, action_pos_emb):
    B = x.size
    x2d = x.reshape(B // GROW, GROW)
    out = _make_sc_gather(B)(x2d, action_emb, action_pos_emb)
    return out.reshape(*x.shape, D)
